# baseline (speedup 1.0000x reference)
import functools

import jax
import jax.numpy as jnp
from jax import lax
from jax.experimental import pallas as pl
from jax.experimental.pallas import tpu as pltpu
from jax.experimental.pallas import tpu_sc as plsc

B = 16384
DIM = 32
RAW_DIM = 96
OUT_DIM = 64
NC = 2
NS = 16
NW = NC * NS
BPW = B // NW


def _mesh():
    return plsc.VectorSubcoreMesh(core_axis_name="c", subcore_axis_name="s")


def _sc_pack3(pt_cat, pt_brand, pt_store):
    vs = [p.shape[1] for p in (pt_cat, pt_brand, pt_store)]

    @functools.partial(
        pl.kernel,
        mesh=_mesh(),
        out_type=[
            jax.ShapeDtypeStruct(((v // 128) * 32, 4 * DIM), jnp.float32)
            for v in vs
        ],
        compiler_params=pltpu.CompilerParams(needs_layout_passes=False),
        scratch_types=[
            pltpu.VMEM((DIM, 128), jnp.float32),
            pltpu.VMEM((DIM, 128), jnp.float32),
            pltpu.VMEM((32, 4 * DIM), jnp.float32),
            pltpu.VMEM((32, 4 * DIM), jnp.float32),
            pltpu.SemaphoreType.DMA,
            pltpu.SemaphoreType.DMA,
            pltpu.SemaphoreType.DMA,
            pltpu.SemaphoreType.DMA,
        ],
    )
    def k(tc_, tb_, ts_, oc, ob, osr, w0, w1, p0, p1, si0, si1, so0, so1):
        wid = lax.axis_index("s") * NC + lax.axis_index("c")
        iota = lax.iota(jnp.int32, 16)

        def shuffle(win, pout):
            for j in range(32):
                for q in range(8):
                    vals = plsc.load_gather(
                        win,
                        [
                            iota + 16 * (q % 2),
                            jnp.full((16,), 4 * j + q // 2, jnp.int32),
                        ],
                    )
                    pout[j, pl.ds(16 * q, 16)] = vals

        for t_hbm, o_hbm in ((tc_, oc), (tb_, ob), (ts_, osr)):
            v = t_hbm.shape[1]
            ntot = v // 128
            per = -(-ntot // NW)
            lo = wid * per
            n = jnp.clip(ntot - lo, 0, per)

            def src_off(i):
                return pl.multiple_of((lo + i) * 128, 128)

            def issue_in(i, buf, sem):
                pltpu.async_copy(
                    t_hbm.at[:, pl.ds(src_off(i), 128)], buf, sem
                )

            def wait_in(buf, sem):
                pltpu.make_async_copy(
                    t_hbm.at[:, pl.ds(0, 128)], buf, sem
                ).wait()

            def issue_out(i, buf, sem):
                dst = pl.multiple_of((lo + i) * 32, 32)
                pltpu.async_copy(buf, o_hbm.at[pl.ds(dst, 32)], sem)

            def wait_out(i, buf, sem):
                pltpu.make_async_copy(
                    buf, o_hbm.at[pl.ds(0, 32)], sem
                ).wait()

            @pl.when(n > 0)
            def _():
                issue_in(0, w0, si0)

            def slot(i, w, p, si_a, si_b, w_other, so, first_pair):
                @pl.when(i < n)
                def _():
                    @pl.when(i + 1 < n)
                    def _():
                        issue_in(i + 1, w_other, si_b)

                    wait_in(w, si_a)
                    if not first_pair:
                        @pl.when(i >= 2)
                        def _():
                            wait_out(i - 2, p, so)

                    shuffle(w, p)
                    issue_out(i, p, so)

            @pl.loop(0, per, step=2)
            def _(i):
                slot(i, w0, p0, si0, si1, w1, so0, False)
                slot(i + 1, w1, p1, si1, si0, w0, so1, False)

            @pl.when(n >= 1)
            def _():
                wait_out(0, p0, so0)

            @pl.when(n >= 2)
            def _():
                wait_out(1, p1, so1)

    return k(pt_cat, pt_brand, pt_store)


def _sc_gather3(i4_cat, i4_brand, i4_store, p_cat, p_brand, p_store):
    @functools.partial(
        pl.kernel,
        mesh=_mesh(),
        out_type=jax.ShapeDtypeStruct((B, 3 * 4 * DIM), jnp.float32),
        scratch_types=[
            pltpu.VMEM((BPW,), jnp.int32),
            pltpu.VMEM((BPW, 4 * DIM), jnp.float32),
            pltpu.SemaphoreType.DMA,
        ],
    )
    def k(ci, bi, si, pc, pb, ps, xo, idx_v, rows_v, sem):
        wid = lax.axis_index("s") * NC + lax.axis_index("c")
        base = wid * BPW
        for t, (i_hbm, t_hbm) in enumerate(
            ((ci, pc), (bi, pb), (si, ps))
        ):
            pltpu.sync_copy(i_hbm.at[pl.ds(base, BPW)], idx_v)
            pltpu.async_copy(t_hbm.at[idx_v], rows_v, sem).wait()
            pltpu.sync_copy(
                rows_v, xo.at[pl.ds(base, BPW), pl.ds(t * 4 * DIM, 4 * DIM)]
            )

    return k(i4_cat, i4_brand, i4_store, p_cat, p_brand, p_store)


BM = 2048


def _tc_project(x, offs, tails, covs, Wt, b2):
    lane_group = 4 * DIM

    def body(x_ref, o_ref, tc_ref, tb_ref, ts_ref, w_ref, bias_ref, out_ref):
        sel = []
        for t, t_ref in enumerate((tc_ref, tb_ref, ts_ref)):
            off = jnp.broadcast_to(o_ref[:, t : t + 1], (BM, lane_group))
            grp = lax.broadcasted_iota(jnp.int32, (BM, lane_group), 1) // DIM
            xm = jnp.where(
                grp == off, x_ref[:, t * lane_group : (t + 1) * lane_group], 0.0
            )
            g = (
                xm[:, 0:DIM]
                + xm[:, DIM : 2 * DIM]
                + xm[:, 2 * DIM : 3 * DIM]
                + xm[:, 3 * DIM : 4 * DIM]
            )
            tv = t_ref.shape[0]
            idx = o_ref[:, 4 + t : 5 + t]
            rel = jnp.broadcast_to(idx - covs[t], (BM, tv))
            oh = jnp.where(
                lax.broadcasted_iota(jnp.int32, (BM, tv), 1) == rel, 1.0, 0.0
            )
            y_tail = jnp.dot(oh, t_ref[...], preferred_element_type=jnp.float32)
            ok = jnp.where(
                jnp.broadcast_to(idx, (BM, DIM)) < covs[t], 1.0, 0.0
            )
            sel.append(g * ok + y_tail)
        xs = jnp.concatenate(sel, axis=1)
        y = jnp.dot(xs, w_ref[...], preferred_element_type=jnp.float32)
        out_ref[...] = jnp.maximum(y + bias_ref[...], 0.0)

    tv_c, tv_b, tv_s = (t.shape[0] for t in tails)
    return pl.pallas_call(
        body,
        grid=(B // BM,),
        in_specs=[
            pl.BlockSpec((BM, 3 * 4 * DIM), lambda i: (i, 0)),
            pl.BlockSpec((BM, 8), lambda i: (i, 0)),
            pl.BlockSpec((tv_c, DIM), lambda i: (0, 0)),
            pl.BlockSpec((tv_b, DIM), lambda i: (0, 0)),
            pl.BlockSpec((tv_s, DIM), lambda i: (0, 0)),
            pl.BlockSpec((RAW_DIM, OUT_DIM), lambda i: (0, 0)),
            pl.BlockSpec((1, OUT_DIM), lambda i: (0, 0)),
        ],
        out_specs=pl.BlockSpec((BM, OUT_DIM), lambda i: (i, 0)),
        out_shape=jax.ShapeDtypeStruct((B, OUT_DIM), jnp.float32),
        compiler_params=pltpu.CompilerParams(
            dimension_semantics=("parallel",)
        ),
    )(x, offs, *tails, Wt, b2)


def kernel(category, brand, store, emb_category, emb_brand, emb_store, W, b):
    ci = category.astype(jnp.int32)
    bi = brand.astype(jnp.int32)
    si = store.astype(jnp.int32)
    p_cat, p_brand, p_store = _sc_pack3(
        emb_category.T, emb_brand.T, emb_store.T
    )
    covs = tuple((e.shape[0] // 128) * 128
                 for e in (emb_category, emb_brand, emb_store))
    i4 = [
        jnp.minimum(idx >> 2, cov // 4 - 1)
        for idx, cov in zip((ci, bi, si), covs)
    ]
    x = _sc_gather3(i4[0], i4[1], i4[2], p_cat, p_brand, p_store)
    z = jnp.zeros_like(ci)
    offs = jnp.stack([ci & 3, bi & 3, si & 3, z, ci, bi, si, z], axis=1)
    tails = (
        emb_category[covs[0] :],
        emb_brand[covs[1] :],
        emb_store[covs[2] :],
    )
    Wt = W.T
    b2 = b.reshape(1, OUT_DIM)
    return _tc_project(x, offs, tails, covs, Wt, b2)


# speedup vs baseline: 1.4213x; 1.4213x over previous
import functools

import jax
import jax.numpy as jnp
from jax import lax
from jax.experimental import pallas as pl
from jax.experimental.pallas import tpu as pltpu
from jax.experimental.pallas import tpu_sc as plsc

B = 16384
DIM = 32
RAW_DIM = 96
OUT_DIM = 64
NC = 2
NS = 16
NW = NC * NS
BPW = B // NW


def _mesh():
    return plsc.VectorSubcoreMesh(core_axis_name="c", subcore_axis_name="s")


def _sc_pack3(pt_cat, pt_brand, pt_store):
    vs = [p.shape[1] for p in (pt_cat, pt_brand, pt_store)]

    @functools.partial(
        pl.kernel,
        mesh=_mesh(),
        out_type=[
            jax.ShapeDtypeStruct(((v // 128) * 32, 4 * DIM), jnp.float32)
            for v in vs
        ],
        compiler_params=pltpu.CompilerParams(needs_layout_passes=False),
        scratch_types=[
            pltpu.VMEM((DIM, 128), jnp.float32),
            pltpu.VMEM((DIM, 128), jnp.float32),
            pltpu.VMEM((32, 4 * DIM), jnp.float32),
            pltpu.VMEM((32, 4 * DIM), jnp.float32),
            pltpu.SemaphoreType.DMA,
            pltpu.SemaphoreType.DMA,
            pltpu.SemaphoreType.DMA,
            pltpu.SemaphoreType.DMA,
        ],
    )
    def k(tc_, tb_, ts_, oc, ob, osr, w0, w1, p0, p1, si0, si1, so0, so1):
        wid = lax.axis_index("s") * NC + lax.axis_index("c")
        iota = lax.iota(jnp.int32, 16)

        def shuffle(win, pout):
            for j0 in range(0, 32, 4):
                vals = [
                    plsc.load_gather(
                        win,
                        [
                            iota + 16 * (q % 2),
                            jnp.full((16,), 4 * j + q // 2, jnp.int32),
                        ],
                    )
                    for j in range(j0, j0 + 4)
                    for q in range(8)
                ]
                for k, j in enumerate(range(j0, j0 + 4)):
                    for q in range(8):
                        pout[j, pl.ds(16 * q, 16)] = vals[k * 8 + q]

        for t_hbm, o_hbm in ((tc_, oc), (tb_, ob), (ts_, osr)):
            v = t_hbm.shape[1]
            ntot = v // 128
            per = -(-ntot // NW)
            lo = wid * per
            n = jnp.clip(ntot - lo, 0, per)

            def src_off(i):
                return pl.multiple_of((lo + i) * 128, 128)

            def issue_in(i, buf, sem):
                pltpu.async_copy(
                    t_hbm.at[:, pl.ds(src_off(i), 128)], buf, sem
                )

            def wait_in(buf, sem):
                pltpu.make_async_copy(
                    t_hbm.at[:, pl.ds(0, 128)], buf, sem
                ).wait()

            def issue_out(i, buf, sem):
                dst = pl.multiple_of((lo + i) * 32, 32)
                pltpu.async_copy(buf, o_hbm.at[pl.ds(dst, 32)], sem)

            def wait_out(i, buf, sem):
                pltpu.make_async_copy(
                    buf, o_hbm.at[pl.ds(0, 32)], sem
                ).wait()

            @pl.when(n > 0)
            def _():
                issue_in(0, w0, si0)

            def slot(i, w, p, si_a, si_b, w_other, so, first_pair):
                @pl.when(i < n)
                def _():
                    @pl.when(i + 1 < n)
                    def _():
                        issue_in(i + 1, w_other, si_b)

                    wait_in(w, si_a)
                    if not first_pair:
                        @pl.when(i >= 2)
                        def _():
                            wait_out(i - 2, p, so)

                    shuffle(w, p)
                    issue_out(i, p, so)

            @pl.loop(0, per, step=2)
            def _(i):
                slot(i, w0, p0, si0, si1, w1, so0, False)
                slot(i + 1, w1, p1, si1, si0, w0, so1, False)

            @pl.when(n >= 1)
            def _():
                wait_out(0, p0, so0)

            @pl.when(n >= 2)
            def _():
                wait_out(1, p1, so1)

    return k(pt_cat, pt_brand, pt_store)


def _sc_gather3(i4_cat, i4_brand, i4_store, p_cat, p_brand, p_store):
    @functools.partial(
        pl.kernel,
        mesh=_mesh(),
        out_type=jax.ShapeDtypeStruct((B, 3 * 4 * DIM), jnp.float32),
        scratch_types=[
            pltpu.VMEM((BPW,), jnp.int32),
            pltpu.VMEM((BPW, 4 * DIM), jnp.float32),
            pltpu.SemaphoreType.DMA,
        ],
    )
    def k(ci, bi, si, pc, pb, ps, xo, idx_v, rows_v, sem):
        wid = lax.axis_index("s") * NC + lax.axis_index("c")
        base = wid * BPW
        for t, (i_hbm, t_hbm) in enumerate(
            ((ci, pc), (bi, pb), (si, ps))
        ):
            pltpu.sync_copy(i_hbm.at[pl.ds(base, BPW)], idx_v)
            pltpu.async_copy(t_hbm.at[idx_v], rows_v, sem).wait()
            pltpu.sync_copy(
                rows_v, xo.at[pl.ds(base, BPW), pl.ds(t * 4 * DIM, 4 * DIM)]
            )

    return k(i4_cat, i4_brand, i4_store, p_cat, p_brand, p_store)


BM = 2048


def _tc_project(x, offs, tails, covs, Wt, b2):
    lane_group = 4 * DIM

    def body(x_ref, o_ref, tc_ref, tb_ref, ts_ref, w_ref, bias_ref, out_ref):
        sel = []
        for t, t_ref in enumerate((tc_ref, tb_ref, ts_ref)):
            off = jnp.broadcast_to(o_ref[:, t : t + 1], (BM, lane_group))
            grp = lax.broadcasted_iota(jnp.int32, (BM, lane_group), 1) // DIM
            xm = jnp.where(
                grp == off, x_ref[:, t * lane_group : (t + 1) * lane_group], 0.0
            )
            g = (
                xm[:, 0:DIM]
                + xm[:, DIM : 2 * DIM]
                + xm[:, 2 * DIM : 3 * DIM]
                + xm[:, 3 * DIM : 4 * DIM]
            )
            tv = t_ref.shape[0]
            idx = o_ref[:, 4 + t : 5 + t]
            rel = jnp.broadcast_to(idx - covs[t], (BM, tv))
            oh = jnp.where(
                lax.broadcasted_iota(jnp.int32, (BM, tv), 1) == rel, 1.0, 0.0
            )
            y_tail = jnp.dot(oh, t_ref[...], preferred_element_type=jnp.float32)
            ok = jnp.where(
                jnp.broadcast_to(idx, (BM, DIM)) < covs[t], 1.0, 0.0
            )
            sel.append(g * ok + y_tail)
        xs = jnp.concatenate(sel, axis=1)
        y = jnp.dot(xs, w_ref[...], preferred_element_type=jnp.float32)
        out_ref[...] = jnp.maximum(y + bias_ref[...], 0.0)

    tv_c, tv_b, tv_s = (t.shape[0] for t in tails)
    return pl.pallas_call(
        body,
        grid=(B // BM,),
        in_specs=[
            pl.BlockSpec((BM, 3 * 4 * DIM), lambda i: (i, 0)),
            pl.BlockSpec((BM, 8), lambda i: (i, 0)),
            pl.BlockSpec((tv_c, DIM), lambda i: (0, 0)),
            pl.BlockSpec((tv_b, DIM), lambda i: (0, 0)),
            pl.BlockSpec((tv_s, DIM), lambda i: (0, 0)),
            pl.BlockSpec((RAW_DIM, OUT_DIM), lambda i: (0, 0)),
            pl.BlockSpec((1, OUT_DIM), lambda i: (0, 0)),
        ],
        out_specs=pl.BlockSpec((BM, OUT_DIM), lambda i: (i, 0)),
        out_shape=jax.ShapeDtypeStruct((B, OUT_DIM), jnp.float32),
        compiler_params=pltpu.CompilerParams(
            dimension_semantics=("parallel",)
        ),
    )(x, offs, *tails, Wt, b2)


def kernel(category, brand, store, emb_category, emb_brand, emb_store, W, b):
    ci = category.astype(jnp.int32)
    bi = brand.astype(jnp.int32)
    si = store.astype(jnp.int32)
    p_cat, p_brand, p_store = _sc_pack3(
        emb_category.T, emb_brand.T, emb_store.T
    )
    covs = tuple((e.shape[0] // 128) * 128
                 for e in (emb_category, emb_brand, emb_store))
    i4 = [
        jnp.minimum(idx >> 2, cov // 4 - 1)
        for idx, cov in zip((ci, bi, si), covs)
    ]
    x = _sc_gather3(i4[0], i4[1], i4[2], p_cat, p_brand, p_store)
    z = jnp.zeros_like(ci)
    offs = jnp.stack([ci & 3, bi & 3, si & 3, z, ci, bi, si, z], axis=1)
    tails = (
        emb_category[covs[0] :],
        emb_brand[covs[1] :],
        emb_store[covs[2] :],
    )
    Wt = W.T
    b2 = b.reshape(1, OUT_DIM)
    return _tc_project(x, offs, tails, covs, Wt, b2)
